# baseline (device time: 68438 ns/iter reference)
import jax
import jax.numpy as jnp
from jax import lax
from jax.experimental import pallas as pl
from jax.experimental.pallas import tpu as pltpu

N_DEV = 8
S = 1024
H = 8
D = 128
HD = H * D
W = 128
G = 32
E = G + 2 * W
SCALE = 0.08838834764831843
NEG = -1e9


def kernel(x, Wq, K_ext, V_ext, Wo):
    def body(x_ref, wq_ref, k_ref, v_ref, wo_ref, out_ref,
             khalo, vhalo, qg_buf, c_stage, ml_stage, c_all, ml_all,
             hsend, hrecv, bsend, brecv, psend, precv):
        i = lax.axis_index("i")

        def rdma(src, dst, ssem, rsem, dev):
            return pltpu.make_async_remote_copy(
                src_ref=src, dst_ref=dst, send_sem=ssem, recv_sem=rsem,
                device_id=dev, device_id_type=pltpu.DeviceIdType.MESH)

        @pl.when(i == 0)
        def _():
            khalo[0:G + W] = jnp.zeros((G + W, H, D), jnp.float32)
            vhalo[0:G + W] = jnp.zeros((G + W, H, D), jnp.float32)

        @pl.when(i == N_DEV - 1)
        def _():
            khalo[G + W:E] = jnp.zeros((W, H, D), jnp.float32)
            vhalo[G + W:E] = jnp.zeros((W, H, D), jnp.float32)

        @pl.when(i < N_DEV - 1)
        def _():
            rdma(k_ref.at[0, pl.ds(S - W, W)], khalo.at[pl.ds(G, W)],
                 hsend.at[0], hrecv.at[0], (i + 1,)).start()
            rdma(v_ref.at[0, pl.ds(S - W, W)], vhalo.at[pl.ds(G, W)],
                 hsend.at[1], hrecv.at[1], (i + 1,)).start()

        @pl.when(i > 0)
        def _():
            rdma(k_ref.at[0, pl.ds(0, W)], khalo.at[pl.ds(G + W, W)],
                 hsend.at[2], hrecv.at[2], (i - 1,)).start()
            rdma(v_ref.at[0, pl.ds(0, W)], vhalo.at[pl.ds(G + W, W)],
                 hsend.at[3], hrecv.at[3], (i - 1,)).start()

        q = jnp.dot(x_ref[0], wq_ref[...], preferred_element_type=jnp.float32)

        @pl.when(i == 0)
        def _():
            qg_buf[...] = q[0:G, :]
            for t in range(1, N_DEV):
                rdma(k_ref.at[0, pl.ds(0, G)], khalo.at[pl.ds(0, G)],
                     bsend.at[0, t - 1], brecv.at[0], (t,)).start()
                rdma(v_ref.at[0, pl.ds(0, G)], vhalo.at[pl.ds(0, G)],
                     bsend.at[1, t - 1], brecv.at[1], (t,)).start()
                rdma(qg_buf, qg_buf,
                     bsend.at[2, t - 1], brecv.at[2], (t,)).start()

        @pl.when(i != 0)
        def _():
            rdma(qg_buf, qg_buf, bsend.at[2, 0], brecv.at[2], (0,)).wait_recv()

        def dot_t(a, b):
            return lax.dot_general(a, b, (((1,), (1,)), ((), ())),
                                   preferred_element_type=jnp.float32)

        qg = qg_buf[...]
        ms = []
        ls = []
        for h in range(H):
            s = dot_t(qg[:, h * D:(h + 1) * D], k_ref[0, :, h, :]) * SCALE
            m = jnp.max(s, axis=1)
            w = jnp.exp(s - m[:, None])
            ls.append(jnp.sum(w, axis=1))
            ms.append(m)
            c_stage[h, :, :] = jnp.dot(w, v_ref[0, :, h, :],
                                       preferred_element_type=jnp.float32)
        ml_stage[0:H, :] = jnp.stack(ms)
        ml_stage[H:2 * H, :] = jnp.stack(ls)

        @pl.when(i == 0)
        def _():
            c_all[0] = c_stage[...]
            ml_all[0] = ml_stage[...]

        @pl.when(i != 0)
        def _():
            rdma(c_stage, c_all.at[i], psend.at[0], precv.at[0, i], (0,)).start()
            rdma(ml_stage, ml_all.at[i], psend.at[1], precv.at[1, i], (0,)).start()

        @pl.when(i > 0)
        def _():
            rdma(khalo.at[pl.ds(G, W)], khalo.at[pl.ds(G, W)],
                 hsend.at[0], hrecv.at[0], (0,)).wait_recv()
            rdma(vhalo.at[pl.ds(G, W)], vhalo.at[pl.ds(G, W)],
                 hsend.at[1], hrecv.at[1], (0,)).wait_recv()
            rdma(khalo.at[pl.ds(0, G)], khalo.at[pl.ds(0, G)],
                 bsend.at[0, 0], brecv.at[0], (0,)).wait_recv()
            rdma(vhalo.at[pl.ds(0, G)], vhalo.at[pl.ds(0, G)],
                 bsend.at[1, 0], brecv.at[1], (0,)).wait_recv()

        @pl.when(i < N_DEV - 1)
        def _():
            rdma(khalo.at[pl.ds(G + W, W)], khalo.at[pl.ds(G + W, W)],
                 hsend.at[2], hrecv.at[2], (0,)).wait_recv()
            rdma(vhalo.at[pl.ds(G + W, W)], vhalo.at[pl.ds(G + W, W)],
                 hsend.at[3], hrecv.at[3], (0,)).wait_recv()

        kg_all = jnp.where(i == 0, k_ref[0, 0:G], khalo[0:G])
        vg_all = jnp.where(i == 0, v_ref[0, 0:G], vhalo[0:G])

        T = 256
        BANDS = ((0, 384), (128, 512), (384, 512), (640, 384))
        for t in range(S // T):
            b0, Wb = BANDS[t]
            rb = lax.broadcasted_iota(jnp.int32, (T, Wb), 0)
            cb = lax.broadcasted_iota(jnp.int32, (T, Wb), 1)
            mask_band = (jnp.abs((T * t + rb) - (b0 + cb)) <= W) \
                | ((i == 0) & (b0 + cb < G))
            rh_ = lax.broadcasted_iota(jnp.int32, (T, W), 0)
            ch_ = lax.broadcasted_iota(jnp.int32, (T, W), 1)
            ctx_blocks = []
            for h in range(H):
                q_t = q[T * t:T * (t + 1), h * D:(h + 1) * D]
                s_band = jnp.where(mask_band,
                                   dot_t(q_t, k_ref[0, b0:b0 + Wb, h, :]) * SCALE,
                                   NEG)
                pieces = [(s_band, v_ref[0, b0:b0 + Wb, h, :])]
                s_gl = dot_t(q_t, kg_all[:, h, :]) * SCALE
                if t == 0:
                    s_gl = jnp.where(i != 0, s_gl, NEG)
                pieces.append((s_gl, vg_all[:, h, :]))
                if t == 0:
                    s_lh = jnp.where((i > 0) & (rh_ <= ch_),
                                     dot_t(q_t, khalo[G:G + W, h, :]) * SCALE,
                                     NEG)
                    pieces.append((s_lh, vhalo[G:G + W, h, :]))
                if t == S // T - 1:
                    s_rh = jnp.where((i < N_DEV - 1) & (rh_ >= ch_ + W),
                                     dot_t(q_t, khalo[G + W:E, h, :]) * SCALE,
                                     NEG)
                    pieces.append((s_rh, vhalo[G + W:E, h, :]))
                m = jnp.max(pieces[0][0], axis=1)
                for s_p, _ in pieces[1:]:
                    m = jnp.maximum(m, jnp.max(s_p, axis=1))
                denom = jnp.zeros((T,), jnp.float32)
                c = jnp.zeros((T, D), jnp.float32)
                for s_p, v_p in pieces:
                    w_p = jnp.exp(s_p - m[:, None])
                    denom = denom + jnp.sum(w_p, axis=1)
                    c = c + jnp.dot(w_p, v_p, preferred_element_type=jnp.float32)
                ctx_blocks.append(c / denom[:, None])
            ctx_t = jnp.concatenate(ctx_blocks, axis=1)
            out_ref[0, T * t:T * (t + 1), :] = jnp.dot(
                ctx_t, wo_ref[...], preferred_element_type=jnp.float32)

        @pl.when(i == 0)
        def _():
            for j in range(1, N_DEV):
                rdma(c_all.at[j], c_all.at[j],
                     psend.at[0], precv.at[0, j], (j,)).wait_recv()
                rdma(ml_all.at[j], ml_all.at[j],
                     psend.at[1], precv.at[1, j], (j,)).wait_recv()
            M = ml_all[0, 0:H, :]
            for j in range(1, N_DEV):
                M = jnp.maximum(M, ml_all[j, 0:H, :])
            Lacc = jnp.zeros((H, G), jnp.float32)
            Cacc = jnp.zeros((H, G, D), jnp.float32)
            for j in range(N_DEV):
                mlj = ml_all[j]
                alpha = jnp.exp(mlj[0:H, :] - M)
                Lacc = Lacc + mlj[H:2 * H, :] * alpha
                Cacc = Cacc + c_all[j] * alpha[:, :, None]
            ctxg = Cacc / Lacc[:, :, None]
            og = jnp.zeros((G, HD), jnp.float32)
            for h in range(H):
                og = og + jnp.dot(ctxg[h], wo_ref[h * D:(h + 1) * D, :],
                                  preferred_element_type=jnp.float32)
            out_ref[0, 0:G, :] = og

        @pl.when(i < N_DEV - 1)
        def _():
            rdma(k_ref.at[0, pl.ds(S - W, W)], khalo.at[pl.ds(G, W)],
                 hsend.at[0], hrecv.at[0], (i + 1,)).wait_send()
            rdma(v_ref.at[0, pl.ds(S - W, W)], vhalo.at[pl.ds(G, W)],
                 hsend.at[1], hrecv.at[1], (i + 1,)).wait_send()

        @pl.when(i > 0)
        def _():
            rdma(k_ref.at[0, pl.ds(0, W)], khalo.at[pl.ds(G + W, W)],
                 hsend.at[2], hrecv.at[2], (i - 1,)).wait_send()
            rdma(v_ref.at[0, pl.ds(0, W)], vhalo.at[pl.ds(G + W, W)],
                 hsend.at[3], hrecv.at[3], (i - 1,)).wait_send()
            rdma(c_stage, c_all.at[i], psend.at[0], precv.at[0, i], (0,)).wait_send()
            rdma(ml_stage, ml_all.at[i], psend.at[1], precv.at[1, i], (0,)).wait_send()

        @pl.when(i == 0)
        def _():
            for t in range(1, N_DEV):
                rdma(k_ref.at[0, pl.ds(0, G)], khalo.at[pl.ds(0, G)],
                     bsend.at[0, t - 1], brecv.at[0], (t,)).wait_send()
                rdma(v_ref.at[0, pl.ds(0, G)], vhalo.at[pl.ds(0, G)],
                     bsend.at[1, t - 1], brecv.at[1], (t,)).wait_send()
                rdma(qg_buf, qg_buf,
                     bsend.at[2, t - 1], brecv.at[2], (t,)).wait_send()

    return pl.pallas_call(
        body,
        out_shape=jax.ShapeDtypeStruct((1, S, HD), jnp.float32),
        in_specs=[pl.BlockSpec(memory_space=pltpu.VMEM)] * 5,
        out_specs=pl.BlockSpec(memory_space=pltpu.VMEM),
        scratch_shapes=[
            pltpu.VMEM((E, H, D), jnp.float32),
            pltpu.VMEM((E, H, D), jnp.float32),
            pltpu.VMEM((G, HD), jnp.float32),
            pltpu.VMEM((H, G, D), jnp.float32),
            pltpu.VMEM((2 * H, G), jnp.float32),
            pltpu.VMEM((N_DEV, H, G, D), jnp.float32),
            pltpu.VMEM((N_DEV, 2 * H, G), jnp.float32),
            pltpu.SemaphoreType.DMA((4,)),
            pltpu.SemaphoreType.DMA((4,)),
            pltpu.SemaphoreType.DMA((3, N_DEV - 1)),
            pltpu.SemaphoreType.DMA((3,)),
            pltpu.SemaphoreType.DMA((2,)),
            pltpu.SemaphoreType.DMA((2, N_DEV)),
        ],
        compiler_params=pltpu.CompilerParams(
            vmem_limit_bytes=26 * 1024 * 1024,
        ),
    )(x, Wq, K_ext, V_ext, Wo)


# device time: 54400 ns/iter; 1.2581x vs baseline; 1.2581x over previous
import jax
import jax.numpy as jnp
from jax import lax
from jax.experimental import pallas as pl
from jax.experimental.pallas import tpu as pltpu

N_DEV = 8
S = 1024
H = 8
D = 128
HD = H * D
W = 128
G = 32
E = G + 2 * W
SCALE = 0.08838834764831843
NEG = -1e9


def kernel(x, Wq, K_ext, V_ext, Wo):
    x2 = x.reshape(S, HD)
    K2 = K_ext.reshape(S, HD)
    V2 = V_ext.reshape(S, HD)

    def body(x_ref, wq_ref, k_ref, v_ref, wo_ref, out_ref,
             khalo, vhalo, qg_buf, c_stage, ml_stage, c_all, ml_all,
             hsend, hrecv, bsend, brecv, psend, precv):
        i = lax.axis_index("i")

        def rdma(src, dst, ssem, rsem, dev):
            return pltpu.make_async_remote_copy(
                src_ref=src, dst_ref=dst, send_sem=ssem, recv_sem=rsem,
                device_id=dev, device_id_type=pltpu.DeviceIdType.MESH)

        @pl.when(i == 0)
        def _():
            khalo[0:G + W, :] = jnp.zeros((G + W, HD), jnp.float32)
            vhalo[0:G + W, :] = jnp.zeros((G + W, HD), jnp.float32)

        @pl.when(i == N_DEV - 1)
        def _():
            khalo[G + W:E, :] = jnp.zeros((W, HD), jnp.float32)
            vhalo[G + W:E, :] = jnp.zeros((W, HD), jnp.float32)

        @pl.when(i < N_DEV - 1)
        def _():
            rdma(k_ref.at[pl.ds(S - W, W)], khalo.at[pl.ds(G, W)],
                 hsend.at[0], hrecv.at[0], (i + 1,)).start()
            rdma(v_ref.at[pl.ds(S - W, W)], vhalo.at[pl.ds(G, W)],
                 hsend.at[1], hrecv.at[1], (i + 1,)).start()

        @pl.when(i > 0)
        def _():
            rdma(k_ref.at[pl.ds(0, W)], khalo.at[pl.ds(G + W, W)],
                 hsend.at[2], hrecv.at[2], (i - 1,)).start()
            rdma(v_ref.at[pl.ds(0, W)], vhalo.at[pl.ds(G + W, W)],
                 hsend.at[3], hrecv.at[3], (i - 1,)).start()

        q = jnp.dot(x_ref[...], wq_ref[...], preferred_element_type=jnp.float32)

        @pl.when(i == 0)
        def _():
            qg_buf[...] = q[0:G, :]
            for t in range(1, N_DEV):
                rdma(k_ref.at[pl.ds(0, G)], khalo.at[pl.ds(0, G)],
                     bsend.at[0, t - 1], brecv.at[0], (t,)).start()
                rdma(v_ref.at[pl.ds(0, G)], vhalo.at[pl.ds(0, G)],
                     bsend.at[1, t - 1], brecv.at[1], (t,)).start()
                rdma(qg_buf, qg_buf,
                     bsend.at[2, t - 1], brecv.at[2], (t,)).start()

        @pl.when(i != 0)
        def _():
            rdma(qg_buf, qg_buf, bsend.at[2, 0], brecv.at[2], (0,)).wait_recv()

        def dot_t(a, b):
            return lax.dot_general(a, b, (((1,), (1,)), ((), ())),
                                   preferred_element_type=jnp.float32)

        qg = qg_buf[...]
        ms = []
        ls = []
        for h in range(H):
            sl = slice(h * D, (h + 1) * D)
            s = dot_t(qg[:, sl], k_ref[:, sl]) * SCALE
            m = jnp.max(s, axis=1)
            w = jnp.exp(s - m[:, None])
            ls.append(jnp.sum(w, axis=1))
            ms.append(m)
            c_stage[h, :, :] = jnp.dot(w, v_ref[:, sl],
                                       preferred_element_type=jnp.float32)
        ml_stage[0:H, :] = jnp.stack(ms)
        ml_stage[H:2 * H, :] = jnp.stack(ls)

        @pl.when(i == 0)
        def _():
            c_all[0] = c_stage[...]
            ml_all[0] = ml_stage[...]

        @pl.when(i != 0)
        def _():
            rdma(c_stage, c_all.at[i], psend.at[0], precv.at[0, i], (0,)).start()
            rdma(ml_stage, ml_all.at[i], psend.at[1], precv.at[1, i], (0,)).start()

        @pl.when(i > 0)
        def _():
            rdma(khalo.at[pl.ds(G, W)], khalo.at[pl.ds(G, W)],
                 hsend.at[0], hrecv.at[0], (0,)).wait_recv()
            rdma(vhalo.at[pl.ds(G, W)], vhalo.at[pl.ds(G, W)],
                 hsend.at[1], hrecv.at[1], (0,)).wait_recv()
            rdma(khalo.at[pl.ds(0, G)], khalo.at[pl.ds(0, G)],
                 bsend.at[0, 0], brecv.at[0], (0,)).wait_recv()
            rdma(vhalo.at[pl.ds(0, G)], vhalo.at[pl.ds(0, G)],
                 bsend.at[1, 0], brecv.at[1], (0,)).wait_recv()

        @pl.when(i < N_DEV - 1)
        def _():
            rdma(khalo.at[pl.ds(G + W, W)], khalo.at[pl.ds(G + W, W)],
                 hsend.at[2], hrecv.at[2], (0,)).wait_recv()
            rdma(vhalo.at[pl.ds(G + W, W)], vhalo.at[pl.ds(G + W, W)],
                 hsend.at[3], hrecv.at[3], (0,)).wait_recv()

        kg_all = jnp.where(i == 0, k_ref[0:G, :], khalo[0:G, :])
        vg_all = jnp.where(i == 0, v_ref[0:G, :], vhalo[0:G, :])

        T = 256
        BANDS = ((0, 384), (128, 512), (384, 512), (640, 384))
        for t in range(S // T):
            b0, Wb = BANDS[t]
            rb = lax.broadcasted_iota(jnp.int32, (T, Wb), 0)
            cb = lax.broadcasted_iota(jnp.int32, (T, Wb), 1)
            mask_band = (jnp.abs((T * t + rb) - (b0 + cb)) <= W) \
                | ((i == 0) & (b0 + cb < G))
            rh_ = lax.broadcasted_iota(jnp.int32, (T, W), 0)
            ch_ = lax.broadcasted_iota(jnp.int32, (T, W), 1)
            ctx_blocks = []
            for h in range(H):
                sl = slice(h * D, (h + 1) * D)
                q_t = q[T * t:T * (t + 1), sl]
                s_band = jnp.where(mask_band,
                                   dot_t(q_t, k_ref[b0:b0 + Wb, sl]) * SCALE,
                                   NEG)
                pieces = [(s_band, v_ref, slice(b0, b0 + Wb))]
                s_gl = dot_t(q_t, kg_all[:, sl]) * SCALE
                if t == 0:
                    s_gl = jnp.where(i != 0, s_gl, NEG)
                pieces.append((s_gl, None, None))
                if t == 0:
                    s_lh = jnp.where((i > 0) & (rh_ <= ch_),
                                     dot_t(q_t, khalo[G:G + W, sl]) * SCALE,
                                     NEG)
                    pieces.append((s_lh, vhalo, slice(G, G + W)))
                if t == S // T - 1:
                    s_rh = jnp.where((i < N_DEV - 1) & (rh_ >= ch_ + W),
                                     dot_t(q_t, khalo[G + W:E, sl]) * SCALE,
                                     NEG)
                    pieces.append((s_rh, vhalo, slice(G + W, E)))
                m = jnp.max(pieces[0][0], axis=1)
                for s_p, _, _ in pieces[1:]:
                    m = jnp.maximum(m, jnp.max(s_p, axis=1))
                denom = jnp.zeros((T,), jnp.float32)
                c = jnp.zeros((T, D), jnp.float32)
                for s_p, v_src, v_sl in pieces:
                    w_p = jnp.exp(s_p - m[:, None])
                    denom = denom + jnp.sum(w_p, axis=1)
                    vb = vg_all[:, sl] if v_src is None else v_src[v_sl, sl]
                    c = c + jnp.dot(w_p, vb, preferred_element_type=jnp.float32)
                ctx_blocks.append(c / denom[:, None])
            ctx_t = jnp.concatenate(ctx_blocks, axis=1)
            out_ref[T * t:T * (t + 1), :] = jnp.dot(
                ctx_t, wo_ref[...], preferred_element_type=jnp.float32)

        @pl.when(i == 0)
        def _():
            for j in range(1, N_DEV):
                rdma(c_all.at[j], c_all.at[j],
                     psend.at[0], precv.at[0, j], (j,)).wait_recv()
                rdma(ml_all.at[j], ml_all.at[j],
                     psend.at[1], precv.at[1, j], (j,)).wait_recv()
            M = ml_all[0, 0:H, :]
            for j in range(1, N_DEV):
                M = jnp.maximum(M, ml_all[j, 0:H, :])
            Lacc = jnp.zeros((H, G), jnp.float32)
            Cacc = jnp.zeros((H, G, D), jnp.float32)
            for j in range(N_DEV):
                mlj = ml_all[j]
                alpha = jnp.exp(mlj[0:H, :] - M)
                Lacc = Lacc + mlj[H:2 * H, :] * alpha
                Cacc = Cacc + c_all[j] * alpha[:, :, None]
            ctxg = Cacc / Lacc[:, :, None]
            og = jnp.zeros((G, HD), jnp.float32)
            for h in range(H):
                og = og + jnp.dot(ctxg[h], wo_ref[h * D:(h + 1) * D, :],
                                  preferred_element_type=jnp.float32)
            out_ref[0:G, :] = og

        @pl.when(i < N_DEV - 1)
        def _():
            rdma(k_ref.at[pl.ds(S - W, W)], khalo.at[pl.ds(G, W)],
                 hsend.at[0], hrecv.at[0], (i + 1,)).wait_send()
            rdma(v_ref.at[pl.ds(S - W, W)], vhalo.at[pl.ds(G, W)],
                 hsend.at[1], hrecv.at[1], (i + 1,)).wait_send()

        @pl.when(i > 0)
        def _():
            rdma(k_ref.at[pl.ds(0, W)], khalo.at[pl.ds(G + W, W)],
                 hsend.at[2], hrecv.at[2], (i - 1,)).wait_send()
            rdma(v_ref.at[pl.ds(0, W)], vhalo.at[pl.ds(G + W, W)],
                 hsend.at[3], hrecv.at[3], (i - 1,)).wait_send()
            rdma(c_stage, c_all.at[i], psend.at[0], precv.at[0, i], (0,)).wait_send()
            rdma(ml_stage, ml_all.at[i], psend.at[1], precv.at[1, i], (0,)).wait_send()

        @pl.when(i == 0)
        def _():
            for t in range(1, N_DEV):
                rdma(k_ref.at[pl.ds(0, G)], khalo.at[pl.ds(0, G)],
                     bsend.at[0, t - 1], brecv.at[0], (t,)).wait_send()
                rdma(v_ref.at[pl.ds(0, G)], vhalo.at[pl.ds(0, G)],
                     bsend.at[1, t - 1], brecv.at[1], (t,)).wait_send()
                rdma(qg_buf, qg_buf,
                     bsend.at[2, t - 1], brecv.at[2], (t,)).wait_send()

    out2 = pl.pallas_call(
        body,
        out_shape=jax.ShapeDtypeStruct((S, HD), jnp.float32),
        in_specs=[pl.BlockSpec(memory_space=pltpu.VMEM)] * 5,
        out_specs=pl.BlockSpec(memory_space=pltpu.VMEM),
        scratch_shapes=[
            pltpu.VMEM((E, HD), jnp.float32),
            pltpu.VMEM((E, HD), jnp.float32),
            pltpu.VMEM((G, HD), jnp.float32),
            pltpu.VMEM((H, G, D), jnp.float32),
            pltpu.VMEM((2 * H, G), jnp.float32),
            pltpu.VMEM((N_DEV, H, G, D), jnp.float32),
            pltpu.VMEM((N_DEV, 2 * H, G), jnp.float32),
            pltpu.SemaphoreType.DMA((4,)),
            pltpu.SemaphoreType.DMA((4,)),
            pltpu.SemaphoreType.DMA((3, N_DEV - 1)),
            pltpu.SemaphoreType.DMA((3,)),
            pltpu.SemaphoreType.DMA((2,)),
            pltpu.SemaphoreType.DMA((2, N_DEV)),
        ],
        compiler_params=pltpu.CompilerParams(
            vmem_limit_bytes=26 * 1024 * 1024,
        ),
    )(x2, Wq, K2, V2, Wo)
    return out2.reshape(1, S, HD)
